# baseline (speedup 1.0000x reference)
import functools

import jax
import jax.numpy as jnp
from jax import lax
from jax.experimental import pallas as pl
from jax.experimental.pallas import tpu as pltpu
from jax.experimental.pallas import tpu_sc as plsc

D = 64
SCALE = 8.0
BPC = 2
SPLITS = ((0, 128), (128, 72))


@functools.lru_cache(maxsize=None)
def _make_gather(NB, T):
    info = plsc.get_sparse_core_info()
    NC, NS, L = info.num_cores, info.num_subcores, info.num_lanes
    NW = NC * NS
    assert NB % (NW * BPC) == 0
    PB = NB // NW
    NIT = PB // BPC
    mesh = plsc.VectorSubcoreMesh(core_axis_name="c", subcore_axis_name="s")

    @functools.partial(
        pl.kernel,
        mesh=mesh,
        compiler_params=pltpu.CompilerParams(use_tc_tiling_on_sc=False),
        out_type=jax.ShapeDtypeStruct((NB, T, D), jnp.float32),
        scratch_types=[
            pltpu.VMEM((2, BPC, T), jnp.int32),
            pltpu.VMEM((2, BPC, T, D), jnp.float32),
            pltpu.SemaphoreType.DMA,
            pltpu.SemaphoreType.DMA,
        ],
    )
    def k(idx_hbm, table_hbm, out_hbm, idx_v, rows_v, gsem, osem):
        wid = lax.axis_index("s") * NC + lax.axis_index("c")
        base = wid * PB

        def gather_copies(g, b):
            copies = []
            for i in range(BPC):
                for (o, n) in SPLITS:
                    copies.append(
                        pltpu.make_async_copy(
                            table_hbm.at[idx_v.at[b].at[i].at[pl.ds(o, n)]],
                            rows_v.at[b].at[i].at[pl.ds(o, n)],
                            gsem,
                        )
                    )
            return copies

        def fire_gather(g, b):
            pltpu.sync_copy(idx_hbm.at[pl.ds(base + g * BPC, BPC)], idx_v.at[b])
            for c in gather_copies(g, b):
                c.start()

        def wait_gather(g, b):
            for c in gather_copies(g, b):
                c.wait()

        fire_gather(0, 0)

        def stage(g, b):
            wait_gather(g, b)

            @pl.when(g > 0)
            def _():
                pltpu.make_async_copy(
                    rows_v.at[1 - b],
                    out_hbm.at[pl.ds(base + (g - 1) * BPC, BPC)],
                    osem,
                ).wait()

            fire_gather(lax.rem(g + 1, NIT), 1 - b)

            def scale_rows(t, carry):
                for i in range(BPC):
                    for c in range(D // L):
                        sl = pl.ds(c * L, L)
                        rows_v[b, i, t, sl] = rows_v[b, i, t, sl] * SCALE
                return carry

            lax.fori_loop(0, T, scale_rows, 0)

            pltpu.async_copy(
                rows_v.at[b], out_hbm.at[pl.ds(base + g * BPC, BPC)], osem
            )

        def outer(i2, carry):
            for b in range(2):
                stage(i2 * 2 + b, b)
            return carry

        lax.fori_loop(0, NIT // 2, outer, 0)

        wait_gather(0, 0)
        pltpu.make_async_copy(
            rows_v.at[1], out_hbm.at[pl.ds(base + (NIT - 1) * BPC, BPC)], osem
        ).wait()

    return k


def kernel(token_ids, table):
    NB, T = token_ids.shape
    return _make_gather(NB, T)(token_ids, table)
